# baseline (speedup 1.0000x reference)
import jax
import jax.numpy as jnp
from jax.experimental import pallas as pl
from jax.experimental.pallas import tpu as pltpu

_HI = jax.lax.Precision.HIGHEST

_SELU_SCALE = 1.0507009873554804934193349852946
_SELU_ALPHA = 1.6732632423543772848170429916717


def _selu(v):
    return _SELU_SCALE * jnp.where(v > 0, v, _SELU_ALPHA * (jnp.exp(v) - 1.0))


def _fused_kernel(k_ref, x_ref, sw0_ref, sb0_ref, sw1_ref, sb1_ref, swo_ref,
                  sbo_ref, mw0_ref, mb0_ref, mw1_ref, mb1_ref, mwo_ref,
                  mbo_ref, deep_ref, mask_ref):
    f32 = jnp.float32
    x = x_ref[...]
    r = x.shape[0]
    d = x.shape[1]

    h = _selu(jnp.dot(x, sw0_ref[...]) + sb0_ref[...])
    h = _selu(jnp.dot(h, sw1_ref[...]) + sb1_ref[...])
    z = jnp.dot(h, swo_ref[...]) + sbo_ref[...]

    kk = k_ref[0].astype(f32)
    ones_d = jnp.ones((d, 1), f32)

    mu = jnp.dot(z, ones_d) * (1.0 / d)
    sig = jnp.sqrt(jnp.maximum(jnp.dot(z * z, ones_d) * (1.0 / d) - mu * mu,
                               0.0))
    lo0 = mu - 0.3 * sig
    hi0 = mu + 4.0 * sig

    h2r = r // 2
    zs = (z[:h2r], z[h2r:])
    los = (lo0[:h2r], lo0[h2r:])
    his = (hi0[:h2r], hi0[h2r:])

    def body(_, carry):
        out = []
        for zh, (lo, hi) in zip(zs, carry):
            mid = 0.5 * (lo + hi)
            cnt = jnp.dot((zh >= mid).astype(f32), ones_d)
            ge = cnt >= kk
            out.append((jnp.where(ge, mid, lo), jnp.where(ge, hi, mid)))
        return tuple(out)

    bounds = jax.lax.fori_loop(0, 12, body, tuple(zip(los, his)))
    lo = jnp.concatenate([bh[0] for bh in bounds], axis=0)

    big = jnp.float32(jnp.inf)
    p1 = z >= lo
    e = jnp.sum(p1.astype(f32), axis=-1, keepdims=True) - kk
    s1 = jnp.min(jnp.where(p1, z, big), axis=-1, keepdims=True)
    s2 = jnp.min(jnp.where(z > s1, z, big), axis=-1, keepdims=True)
    s3 = jnp.min(jnp.where(z > s2, z, big), axis=-1, keepdims=True)
    thr = jnp.where(e <= 0.0, s1, jnp.where(e == 1.0, s2, s3))

    mask = (z >= thr).astype(f32)
    mask_ref[...] = mask

    h2 = jnp.maximum(jnp.dot(x * mask, mw0_ref[...]) + mb0_ref[...], 0.0)
    h2 = jnp.maximum(jnp.dot(h2, mw1_ref[...]) + mb1_ref[...], 0.0)
    logits = jnp.dot(h2, mwo_ref[...]) + mbo_ref[...]
    m = jnp.max(logits, axis=-1, keepdims=True)
    e = jnp.exp(logits - m)
    deep_ref[...] = e / jnp.sum(e, axis=-1, keepdims=True)


def kernel(inputs, sel_W0, sel_b0, sel_W1, sel_b1, sel_Wout, sel_bout,
           mlp_W0, mlp_b0, mlp_W1, mlp_b1, mlp_Wout, mlp_bout, k):
    b, d = inputs.shape
    s0 = sel_W0.shape[1]
    s1 = sel_W1.shape[1]
    m0 = mlp_W0.shape[1]
    m1 = mlp_W1.shape[1]
    mo = mlp_Wout.shape[1]
    blk = 2048 if b % 2048 == 0 else b
    grid = (b // blk,)

    k_arr = jnp.asarray(k, jnp.int32).reshape(1)
    full = lambda shape: pl.BlockSpec(shape, lambda i: (0, 0))

    out = pl.pallas_call(
        _fused_kernel,
        grid=grid,
        in_specs=[
            pl.BlockSpec(memory_space=pltpu.SMEM),
            pl.BlockSpec((blk, d), lambda i: (i, 0)),
            full((d, s0)), full((1, s0)),
            full((s0, s1)), full((1, s1)),
            full((s1, d)), full((1, d)),
            full((d, m0)), full((1, m0)),
            full((m0, m1)), full((1, m1)),
            full((m1, mo)), full((1, mo)),
        ],
        out_specs=[
            pl.BlockSpec((blk, mo), lambda i: (i, 0)),
            pl.BlockSpec((blk, d), lambda i: (i, 0)),
        ],
        out_shape=[
            jax.ShapeDtypeStruct((b, mo), jnp.float32),
            jax.ShapeDtypeStruct((b, d), jnp.float32),
        ],
        compiler_params=pltpu.CompilerParams(
            dimension_semantics=("parallel",)),
    )(k_arr, inputs,
      sel_W0, sel_b0.reshape(1, -1),
      sel_W1, sel_b1.reshape(1, -1),
      sel_Wout, sel_bout.reshape(1, -1),
      mlp_W0, mlp_b0.reshape(1, -1),
      mlp_W1, mlp_b1.reshape(1, -1),
      mlp_Wout, mlp_bout.reshape(1, -1))
    deep_out, masks = out
    return (deep_out, masks)


# speedup vs baseline: 1.0407x; 1.0407x over previous
import jax
import jax.numpy as jnp
from jax.experimental import pallas as pl
from jax.experimental.pallas import tpu as pltpu

_HI = jax.lax.Precision.HIGHEST

_SELU_SCALE = 1.0507009873554804934193349852946
_SELU_ALPHA = 1.6732632423543772848170429916717


def _selu(v):
    return _SELU_SCALE * jnp.where(v > 0, v, _SELU_ALPHA * (jnp.exp(v) - 1.0))


def _fused_kernel(k_ref, x_ref, sw0_ref, sb0_ref, sw1_ref, sb1_ref, swo_ref,
                  sbo_ref, mw0_ref, mb0_ref, mw1_ref, mb1_ref, mwo_ref,
                  mbo_ref, deep_ref, mask_ref):
    f32 = jnp.float32
    x = x_ref[...]
    r = x.shape[0]
    d = x.shape[1]

    h = _selu(jnp.dot(x, sw0_ref[...]) + sb0_ref[...])
    h = _selu(jnp.dot(h, sw1_ref[...]) + sb1_ref[...])
    z = jnp.dot(h, swo_ref[...]) + sbo_ref[...]

    kk = k_ref[0].astype(f32)

    ones_d = jnp.ones((d, 1), f32)
    mu = jnp.dot(z, ones_d) * (1.0 / d)
    sig = jnp.sqrt(jnp.maximum(jnp.dot(z * z, ones_d) * (1.0 / d) - mu * mu,
                               0.0))
    lo0 = mu - 0.3 * sig
    hi0 = mu + 4.0 * sig

    def body(_, carry):
        lo, hi = carry
        mid = 0.5 * (lo + hi)
        cnt = jnp.sum((z >= mid).astype(f32), axis=-1, keepdims=True)
        ge = cnt >= kk
        return jnp.where(ge, mid, lo), jnp.where(ge, hi, mid)

    lo, _ = jax.lax.fori_loop(0, 12, body, (lo0, hi0))

    big = jnp.float32(jnp.inf)
    p1 = z >= lo
    e = jnp.sum(p1.astype(f32), axis=-1, keepdims=True) - kk
    s1 = jnp.min(jnp.where(p1, z, big), axis=-1, keepdims=True)
    s2 = jnp.min(jnp.where(z > s1, z, big), axis=-1, keepdims=True)
    s3 = jnp.min(jnp.where(z > s2, z, big), axis=-1, keepdims=True)
    thr = jnp.where(e <= 0.0, s1, jnp.where(e == 1.0, s2, s3))

    mask = (z >= thr).astype(f32)
    mask_ref[...] = mask

    h2 = jnp.maximum(jnp.dot(x * mask, mw0_ref[...]) + mb0_ref[...], 0.0)
    h2 = jnp.maximum(jnp.dot(h2, mw1_ref[...]) + mb1_ref[...], 0.0)
    logits = jnp.dot(h2, mwo_ref[...]) + mbo_ref[...]
    m = jnp.max(logits, axis=-1, keepdims=True)
    e = jnp.exp(logits - m)
    deep_ref[...] = e / jnp.sum(e, axis=-1, keepdims=True)


def kernel(inputs, sel_W0, sel_b0, sel_W1, sel_b1, sel_Wout, sel_bout,
           mlp_W0, mlp_b0, mlp_W1, mlp_b1, mlp_Wout, mlp_bout, k):
    b, d = inputs.shape
    s0 = sel_W0.shape[1]
    s1 = sel_W1.shape[1]
    m0 = mlp_W0.shape[1]
    m1 = mlp_W1.shape[1]
    mo = mlp_Wout.shape[1]
    blk = 2048 if b % 2048 == 0 else b
    grid = (b // blk,)

    k_arr = jnp.asarray(k, jnp.int32).reshape(1)
    full = lambda shape: pl.BlockSpec(shape, lambda i: (0, 0))

    out = pl.pallas_call(
        _fused_kernel,
        grid=grid,
        in_specs=[
            pl.BlockSpec(memory_space=pltpu.SMEM),
            pl.BlockSpec((blk, d), lambda i: (i, 0)),
            full((d, s0)), full((1, s0)),
            full((s0, s1)), full((1, s1)),
            full((s1, d)), full((1, d)),
            full((d, m0)), full((1, m0)),
            full((m0, m1)), full((1, m1)),
            full((m1, mo)), full((1, mo)),
        ],
        out_specs=[
            pl.BlockSpec((blk, mo), lambda i: (i, 0)),
            pl.BlockSpec((blk, d), lambda i: (i, 0)),
        ],
        out_shape=[
            jax.ShapeDtypeStruct((b, mo), jnp.float32),
            jax.ShapeDtypeStruct((b, d), jnp.float32),
        ],
        compiler_params=pltpu.CompilerParams(
            dimension_semantics=("parallel",)),
    )(k_arr, inputs,
      sel_W0, sel_b0.reshape(1, -1),
      sel_W1, sel_b1.reshape(1, -1),
      sel_Wout, sel_bout.reshape(1, -1),
      mlp_W0, mlp_b0.reshape(1, -1),
      mlp_W1, mlp_b1.reshape(1, -1),
      mlp_Wout, mlp_bout.reshape(1, -1))
    deep_out, masks = out
    return (deep_out, masks)


# speedup vs baseline: 1.2228x; 1.1750x over previous
import jax
import jax.numpy as jnp
from jax.experimental import pallas as pl
from jax.experimental.pallas import tpu as pltpu

_HI = jax.lax.Precision.HIGHEST

_SELU_SCALE = 1.0507009873554804934193349852946
_SELU_ALPHA = 1.6732632423543772848170429916717


def _selu(v):
    return _SELU_SCALE * jnp.where(v > 0, v, _SELU_ALPHA * (jnp.exp(v) - 1.0))


def _fused_kernel(k_ref, x_ref, sw0_ref, sb0_ref, sw1_ref, sb1_ref, swo_ref,
                  sbo_ref, mw0_ref, mb0_ref, mw1_ref, mb1_ref, mwo_ref,
                  mbo_ref, deep_ref, mask_ref):
    f32 = jnp.float32
    x = x_ref[...]
    r = x.shape[0]
    d = x.shape[1]

    h = _selu(jnp.dot(x, sw0_ref[...]) + sb0_ref[...])
    h = _selu(jnp.dot(h, sw1_ref[...]) + sb1_ref[...])
    z = jnp.dot(h, swo_ref[...]) + sbo_ref[...]

    kk = k_ref[0].astype(f32)

    mu = jnp.mean(z, axis=-1, keepdims=True)
    sig = jnp.sqrt(jnp.maximum(
        jnp.mean(z * z, axis=-1, keepdims=True) - mu * mu, 0.0))
    lo0 = mu - 0.3 * sig
    hi0 = mu + 4.0 * sig

    def body(_, carry):
        lo, hi = carry
        mid = 0.5 * (lo + hi)
        cnt = jnp.sum((z >= mid).astype(f32), axis=-1, keepdims=True)
        ge = cnt >= kk
        return jnp.where(ge, mid, lo), jnp.where(ge, hi, mid)

    lo, _ = jax.lax.fori_loop(0, 12, body, (lo0, hi0), unroll=True)

    big = jnp.float32(jnp.inf)
    p1 = z >= lo
    e = jnp.sum(p1.astype(f32), axis=-1, keepdims=True) - kk
    s1 = jnp.min(jnp.where(p1, z, big), axis=-1, keepdims=True)
    s2 = jnp.min(jnp.where(z > s1, z, big), axis=-1, keepdims=True)
    s3 = jnp.min(jnp.where(z > s2, z, big), axis=-1, keepdims=True)
    thr = jnp.where(e <= 0.0, s1, jnp.where(e == 1.0, s2, s3))

    mask = (z >= thr).astype(f32)
    mask_ref[...] = mask

    h2 = jnp.maximum(jnp.dot(x * mask, mw0_ref[...]) + mb0_ref[...], 0.0)
    h2 = jnp.maximum(jnp.dot(h2, mw1_ref[...]) + mb1_ref[...], 0.0)
    logits = jnp.dot(h2, mwo_ref[...]) + mbo_ref[...]
    m = jnp.max(logits, axis=-1, keepdims=True)
    e = jnp.exp(logits - m)
    deep_ref[...] = e / jnp.sum(e, axis=-1, keepdims=True)


def kernel(inputs, sel_W0, sel_b0, sel_W1, sel_b1, sel_Wout, sel_bout,
           mlp_W0, mlp_b0, mlp_W1, mlp_b1, mlp_Wout, mlp_bout, k):
    b, d = inputs.shape
    s0 = sel_W0.shape[1]
    s1 = sel_W1.shape[1]
    m0 = mlp_W0.shape[1]
    m1 = mlp_W1.shape[1]
    mo = mlp_Wout.shape[1]
    blk = 2048 if b % 2048 == 0 else b
    grid = (b // blk,)

    k_arr = jnp.asarray(k, jnp.int32).reshape(1)
    full = lambda shape: pl.BlockSpec(shape, lambda i: (0, 0))

    out = pl.pallas_call(
        _fused_kernel,
        grid=grid,
        in_specs=[
            pl.BlockSpec(memory_space=pltpu.SMEM),
            pl.BlockSpec((blk, d), lambda i: (i, 0)),
            full((d, s0)), full((1, s0)),
            full((s0, s1)), full((1, s1)),
            full((s1, d)), full((1, d)),
            full((d, m0)), full((1, m0)),
            full((m0, m1)), full((1, m1)),
            full((m1, mo)), full((1, mo)),
        ],
        out_specs=[
            pl.BlockSpec((blk, mo), lambda i: (i, 0)),
            pl.BlockSpec((blk, d), lambda i: (i, 0)),
        ],
        out_shape=[
            jax.ShapeDtypeStruct((b, mo), jnp.float32),
            jax.ShapeDtypeStruct((b, d), jnp.float32),
        ],
        compiler_params=pltpu.CompilerParams(
            dimension_semantics=("parallel",)),
    )(k_arr, inputs,
      sel_W0, sel_b0.reshape(1, -1),
      sel_W1, sel_b1.reshape(1, -1),
      sel_Wout, sel_bout.reshape(1, -1),
      mlp_W0, mlp_b0.reshape(1, -1),
      mlp_W1, mlp_b1.reshape(1, -1),
      mlp_Wout, mlp_bout.reshape(1, -1))
    deep_out, masks = out
    return (deep_out, masks)
